# Initial kernel scaffold; baseline (speedup 1.0000x reference)
import jax
import jax.numpy as jnp
from jax.experimental import pallas as pl


def kernel(token_embeddings, attention_mask, item_pos_w, layer_w, temporal, ln_w, ln_b):
    raise NotImplementedError("write your pallas kernel here")


# speedup vs baseline: 3.1651x; 3.1651x over previous
import functools

import jax
import jax.numpy as jnp
from jax.experimental import pallas as pl
from jax.experimental.pallas import tpu as pltpu

B = 16
S = 4096
D = 768
MAX_ITEMS = 1366
NUM_LAYERS = 3
EPS = 1e-5
BLK = 512
NBLK = S // BLK


def _prep_kernel(ipw_ref, tmp_ref, lw_ref, mask_ref,
                 c0_ref, c1_ref, c2_ref, start_ref):
    ipt = ipw_ref[...] + tmp_ref[...]
    c0_ref[...] = ipt + lw_ref[0:1, :]
    c1_ref[...] = ipt + lw_ref[1:2, :]
    c2_ref[...] = ipt + lw_ref[2:3, :]
    pos = jax.lax.broadcasted_iota(jnp.int32, (B, S), 1)
    masked_pos = jnp.where(mask_ref[...] > 0, pos, jnp.int32(S))
    start_ref[...] = jnp.min(masked_pos, axis=1, keepdims=True)


def _main_kernel(tok_ref, maskf_ref, cpad_ref, start_ref, lnw_ref, lnb_ref,
                 out_ref):
    b = pl.program_id(0)
    j = pl.program_id(1)
    s0 = start_ref[b, 0]
    off = jnp.clip(j * BLK - s0, -BLK, S - BLK) + BLK
    off8 = pl.multiple_of((off // 8) * 8, 8)
    r = off - off8
    big = cpad_ref[pl.ds(off8, BLK + 8), :]
    shift = jnp.where(r == 0, 0, BLK + 8 - r)
    cw = pltpu.roll(big, shift, 0)[:BLK]
    enh = tok_ref[0] + cw
    mean = jnp.mean(enh, axis=1, keepdims=True)
    cent = enh - mean
    var = jnp.mean(cent * cent, axis=1, keepdims=True)
    normed = cent * jax.lax.rsqrt(var + EPS) * lnw_ref[...] + lnb_ref[...]
    out_ref[0] = normed * maskf_ref[0]


@functools.partial(jax.jit, static_argnames=("interpret",))
def kernel(token_embeddings, attention_mask, item_pos_w, layer_w, temporal,
           ln_w, ln_b, interpret=False):
    mask = attention_mask.astype(jnp.int32)

    c0, c1, c2, start = pl.pallas_call(
        _prep_kernel,
        out_shape=(
            jax.ShapeDtypeStruct((MAX_ITEMS, D), jnp.float32),
            jax.ShapeDtypeStruct((MAX_ITEMS, D), jnp.float32),
            jax.ShapeDtypeStruct((MAX_ITEMS, D), jnp.float32),
            jax.ShapeDtypeStruct((B, 1), jnp.int32),
        ),
        interpret=interpret,
    )(item_pos_w, temporal, layer_w, mask)

    cflat = jnp.stack([c0, c1, c2], axis=1).reshape(MAX_ITEMS * NUM_LAYERS, D)
    cpad = jnp.concatenate(
        [jnp.zeros((BLK, D), jnp.float32), cflat[:S],
         jnp.zeros((8, D), jnp.float32)], axis=0)

    maskf = mask.astype(jnp.float32).reshape(B, S, 1)
    lnw = ln_w.reshape(1, D)
    lnb = ln_b.reshape(1, D)

    out = pl.pallas_call(
        _main_kernel,
        grid=(B, NBLK),
        in_specs=[
            pl.BlockSpec((1, BLK, D), lambda b, j: (b, j, 0)),
            pl.BlockSpec((1, BLK, 1), lambda b, j: (b, j, 0)),
            pl.BlockSpec((S + BLK + 8, D), lambda b, j: (0, 0)),
            pl.BlockSpec(memory_space=pltpu.SMEM),
            pl.BlockSpec((1, D), lambda b, j: (0, 0)),
            pl.BlockSpec((1, D), lambda b, j: (0, 0)),
        ],
        out_specs=pl.BlockSpec((1, BLK, D), lambda b, j: (b, j, 0)),
        out_shape=jax.ShapeDtypeStruct((B, S, D), jnp.float32),
        interpret=interpret,
    )(token_embeddings, maskf, cpad, start, lnw, lnb)
    return out
